# Initial kernel scaffold; baseline (speedup 1.0000x reference)
import jax
import jax.numpy as jnp
from jax.experimental import pallas as pl


def kernel(x, edge_index, W1, a_src, a_dst, b1, gamma1, beta1, W2, b2, gamma2, beta2, Wf, bf):
    raise NotImplementedError("write your pallas kernel here")


# speedup vs baseline: 60.0005x; 60.0005x over previous
import functools

import jax
import jax.numpy as jnp
from jax import lax
from jax.experimental import pallas as pl
from jax.experimental.pallas import tpu as pltpu
from jax.experimental.pallas import tpu_sc as plsc

N = 35000
E = 560000
NP = 35072
EP = 573440
CHUNK = 128
ROWS_PER_TILE = NP // 16

BN = 1000
GRID_N = N // BN


def _k1_body(x_ref, w_ref, avs_ref, avd_ref, h_out, att_out):
    hb = jnp.dot(x_ref[...], w_ref[...], preferred_element_type=jnp.float32)
    h_out[...] = hb
    ts = hb * avs_ref[...]
    td = hb * avd_ref[...]
    as0 = jnp.sum(ts[:, :32], axis=1, keepdims=True)
    as1 = jnp.sum(ts[:, 32:], axis=1, keepdims=True)
    ad0 = jnp.sum(td[:, :32], axis=1, keepdims=True)
    ad1 = jnp.sum(td[:, 32:], axis=1, keepdims=True)
    att_out[...] = jnp.concatenate([as0, as1, ad0, ad1], axis=1)


def _k1(x, w1f, avs, avd):
    return pl.pallas_call(
        _k1_body,
        grid=(GRID_N,),
        in_specs=[
            pl.BlockSpec((BN, 128), lambda i: (i, 0)),
            pl.BlockSpec((128, 64), lambda i: (0, 0)),
            pl.BlockSpec((1, 64), lambda i: (0, 0)),
            pl.BlockSpec((1, 64), lambda i: (0, 0)),
        ],
        out_specs=[
            pl.BlockSpec((BN, 64), lambda i: (i, 0)),
            pl.BlockSpec((BN, 4), lambda i: (i, 0)),
        ],
        out_shape=[
            jax.ShapeDtypeStruct((N, 64), jnp.float32),
            jax.ShapeDtypeStruct((N, 4), jnp.float32),
        ],
    )(x, w1f, avs, avd)


ACH = 1280

def _att_body(ei, att, ex_out, asrcv, adstv, idx, exb):
    c = lax.axis_index("c")
    s = lax.axis_index("s")
    pltpu.sync_copy(att.at[c, 0], asrcv)
    pltpu.sync_copy(att.at[c, 1], adstv)
    tile_base = s * (EP // 16)

    def body(i, carry):
        base = tile_base + i * ACH
        pltpu.sync_copy(ei.at[0, pl.ds(base, ACH)], idx.at[0])
        pltpu.sync_copy(ei.at[1, pl.ds(base, ACH)], idx.at[1])
        for g in range(ACH // 16):
            sv = idx[0, pl.ds(g * 16, 16)]
            dv = idx[1, pl.ds(g * 16, 16)]
            a_s = plsc.load_gather(asrcv, [sv])
            a_d = plsc.load_gather(adstv, [dv])
            e = a_s + a_d
            e = jnp.where(e > 0, e, 0.2 * e)
            exb[pl.ds(g * 16, 16)] = jnp.exp(e)
        pltpu.sync_copy(exb, ex_out.at[c, pl.ds(base, ACH)])
        return carry

    lax.fori_loop(0, EP // 16 // ACH, body, 0)


def _sc_att(ei_pad, att_sc):
    f = pl.kernel(
        _att_body,
        out_type=jax.ShapeDtypeStruct((2, EP), jnp.float32),
        mesh=plsc.VectorSubcoreMesh(core_axis_name="c", subcore_axis_name="s",
                                    num_cores=2, num_subcores=16),
        compiler_params=pltpu.CompilerParams(needs_layout_passes=False, use_tc_tiling_on_sc=False),
        scratch_types=[
            pltpu.VMEM((NP,), jnp.float32),
            pltpu.VMEM((NP,), jnp.float32),
            pltpu.VMEM((2, ACH), jnp.int32),
            pltpu.VMEM((ACH,), jnp.float32),
        ],
    )
    return f(ei_pad, att_sc)


def _agg_body(ei, h_st, exh, z48, agg_out,
              idx, gidx, rows, outb, exv, acc, gsem0, gsem1):
    c = lax.axis_index("c")
    s = lax.axis_index("s")
    pltpu.sync_copy(z48, acc.at[pl.ds(s * ROWS_PER_TILE, ROWS_PER_TILE)])
    plsc.subcore_barrier()

    tile_base = s * (EP // 16)
    iota16 = lax.iota(jnp.int32, 16)
    hoff = c * NP
    degval = jnp.where(c == 0, 1.0, 0.0).astype(jnp.float32)
    oh0 = jnp.where(iota16 == 0, 1.0, 0.0).astype(jnp.float32)
    degvec = jnp.where(iota16 == 1, degval, 0.0).astype(jnp.float32)

    gsems = (gsem0, gsem1)

    def start_chunk(i, b):
        base = tile_base + i * CHUNK
        pltpu.sync_copy(ei.at[0, pl.ds(base, CHUNK)], idx.at[b, 0])
        pltpu.sync_copy(ei.at[1, pl.ds(base, CHUNK)], idx.at[b, 1])
        pltpu.sync_copy(exh.at[c, pl.ds(base, CHUNK)], exv.at[b])
        for g in range(8):
            sv = idx[b, 0, pl.ds(g * 16, 16)]
            gidx[b, pl.ds(g * 16, 16)] = sv + hoff
        pltpu.async_copy(h_st.at[gidx.at[b]], rows.at[b], gsems[b])

    def finish_chunk(b):
        pltpu.make_async_copy(h_st.at[gidx.at[b]], rows.at[b], gsems[b]).wait()
        for g in range(8):
            ex = exv[b, pl.ds(g * 16, 16)]
            for l in range(16):
                el = g * 16 + l
                exs = ex[l]
                outb[b, el, pl.ds(0, 16)] = rows[b, el, pl.ds(0, 16)] * exs
                outb[b, el, pl.ds(16, 16)] = rows[b, el, pl.ds(16, 16)] * exs
                outb[b, el, pl.ds(32, 16)] = oh0 * exs + degvec
        pltpu.sync_copy(outb.at[b], acc.at[idx.at[b, 1]], add=True)

    nch = EP // 16 // CHUNK
    start_chunk(0, 0)

    def body(j, carry):
        i0 = 2 * j
        start_chunk(i0 + 1, 1)
        finish_chunk(0)

        @pl.when(i0 + 2 < nch)
        def _():
            start_chunk(i0 + 2, 0)

        finish_chunk(1)
        return carry

    lax.fori_loop(0, nch // 2, body, 0)
    plsc.subcore_barrier()
    pltpu.sync_copy(acc.at[pl.ds(s * ROWS_PER_TILE, ROWS_PER_TILE)],
                    agg_out.at[c, pl.ds(s * ROWS_PER_TILE, ROWS_PER_TILE)])


def _sc_agg(ei_pad, h_st, exh, z48):
    f = pl.kernel(
        _agg_body,
        out_type=jax.ShapeDtypeStruct((2, NP, 48), jnp.float32),
        mesh=plsc.VectorSubcoreMesh(core_axis_name="c", subcore_axis_name="s",
                                    num_cores=2, num_subcores=16),
        compiler_params=pltpu.CompilerParams(needs_layout_passes=False, use_tc_tiling_on_sc=False),
        scratch_types=[
            pltpu.VMEM((2, 2, CHUNK), jnp.int32),
            pltpu.VMEM((2, CHUNK), jnp.int32),
            pltpu.VMEM((2, CHUNK, 32), jnp.float32),
            pltpu.VMEM((2, CHUNK, 48), jnp.float32),
            pltpu.VMEM((2, CHUNK), jnp.float32),
            pltpu.VMEM_SHARED((NP, 48), jnp.float32),
            pltpu.SemaphoreType.DMA,
            pltpu.SemaphoreType.DMA,
        ],
    )
    return f(ei_pad, h_st, exh, z48)


def _gcn_body(ei, xw, dinv, z32, out,
              dv_tab, idx, rows, outb, acc, gsem0, gsem1):
    c = lax.axis_index("c")
    s = lax.axis_index("s")
    pltpu.sync_copy(z32, acc.at[pl.ds(s * ROWS_PER_TILE, ROWS_PER_TILE)])
    pltpu.sync_copy(dinv, dv_tab)
    plsc.subcore_barrier()

    tile_base = (c * 16 + s) * (EP // 32)
    gsems = (gsem0, gsem1)

    def start_chunk(i, b):
        base = tile_base + i * CHUNK
        pltpu.sync_copy(ei.at[0, pl.ds(base, CHUNK)], idx.at[b, 0])
        pltpu.sync_copy(ei.at[1, pl.ds(base, CHUNK)], idx.at[b, 1])
        pltpu.async_copy(xw.at[idx.at[b, 0]], rows.at[b], gsems[b])

    def finish_chunk(b):
        pltpu.make_async_copy(xw.at[idx.at[b, 0]], rows.at[b], gsems[b]).wait()
        for g in range(8):
            sv = idx[b, 0, pl.ds(g * 16, 16)]
            dv = idx[b, 1, pl.ds(g * 16, 16)]
            nv = plsc.load_gather(dv_tab, [sv]) * plsc.load_gather(dv_tab, [dv])
            for l in range(16):
                el = g * 16 + l
                ns = nv[l]
                outb[b, el, pl.ds(0, 16)] = rows[b, el, pl.ds(0, 16)] * ns
                outb[b, el, pl.ds(16, 16)] = rows[b, el, pl.ds(16, 16)] * ns
        pltpu.sync_copy(outb.at[b], acc.at[idx.at[b, 1]], add=True)

    nch = EP // 32 // CHUNK
    start_chunk(0, 0)

    def body(j, carry):
        i0 = 2 * j
        start_chunk(i0 + 1, 1)
        finish_chunk(0)

        @pl.when(i0 + 2 < nch)
        def _():
            start_chunk(i0 + 2, 0)

        finish_chunk(1)
        return carry

    lax.fori_loop(0, nch // 2, body, 0)
    plsc.subcore_barrier()
    pltpu.sync_copy(acc.at[pl.ds(s * ROWS_PER_TILE, ROWS_PER_TILE)],
                    out.at[c, pl.ds(s * ROWS_PER_TILE, ROWS_PER_TILE)])


def _sc_gcn(ei_pad, xw_pad, dinv_pad, z32):
    f = pl.kernel(
        _gcn_body,
        out_type=jax.ShapeDtypeStruct((2, NP, 32), jnp.float32),
        mesh=plsc.VectorSubcoreMesh(core_axis_name="c", subcore_axis_name="s",
                                    num_cores=2, num_subcores=16),
        compiler_params=pltpu.CompilerParams(needs_layout_passes=False, use_tc_tiling_on_sc=False),
        scratch_types=[
            pltpu.VMEM((NP,), jnp.float32),
            pltpu.VMEM((2, 2, CHUNK), jnp.int32),
            pltpu.VMEM((2, CHUNK, 32), jnp.float32),
            pltpu.VMEM((2, CHUNK, 32), jnp.float32),
            pltpu.VMEM_SHARED((NP, 32), jnp.float32),
            pltpu.SemaphoreType.DMA,
            pltpu.SemaphoreType.DMA,
        ],
    )
    return f(ei_pad, xw_pad, dinv_pad, z32)


def _k2a_body(h_ref, att_ref, agg0, agg1, den0, den1, deg_ref, b1_ref,
              x1_out, ps_out, pss_out, dinv_out):
    att = att_ref[...]
    h = h_ref[...]

    def head(aggr, denr, asl, adl, hsl):
        es = asl + adl
        es = jnp.where(es > 0, es, 0.2 * es)
        exs = jnp.exp(es)
        den = denr[...] + exs
        num = aggr[...] + hsl * exs
        return num / (den + 1e-16)

    a0 = head(agg0, den0, att[:, 0:1], att[:, 2:3], h[:, :32])
    a1 = head(agg1, den1, att[:, 1:2], att[:, 3:4], h[:, 32:])
    x1 = jnp.concatenate([a0, a1], axis=1) + b1_ref[...]
    x1_out[...] = x1
    ps_out[...] = jnp.sum(x1, axis=0, keepdims=True)[None]
    pss_out[...] = jnp.sum(x1 * x1, axis=0, keepdims=True)[None]
    dinv_out[...] = lax.rsqrt(deg_ref[...] + 1.0)


def _k2a(h64, att4, agg0, agg1, den0, den1, deg, b1r):
    return pl.pallas_call(
        _k2a_body,
        grid=(GRID_N,),
        in_specs=[
            pl.BlockSpec((BN, 64), lambda i: (i, 0)),
            pl.BlockSpec((BN, 4), lambda i: (i, 0)),
            pl.BlockSpec((BN, 32), lambda i: (i, 0)),
            pl.BlockSpec((BN, 32), lambda i: (i, 0)),
            pl.BlockSpec((BN, 1), lambda i: (i, 0)),
            pl.BlockSpec((BN, 1), lambda i: (i, 0)),
            pl.BlockSpec((BN, 1), lambda i: (i, 0)),
            pl.BlockSpec((1, 64), lambda i: (0, 0)),
        ],
        out_specs=[
            pl.BlockSpec((BN, 64), lambda i: (i, 0)),
            pl.BlockSpec((1, 1, 64), lambda i: (i, 0, 0)),
            pl.BlockSpec((1, 1, 64), lambda i: (i, 0, 0)),
            pl.BlockSpec((BN, 1), lambda i: (i, 0)),
        ],
        out_shape=[
            jax.ShapeDtypeStruct((N, 64), jnp.float32),
            jax.ShapeDtypeStruct((GRID_N, 1, 64), jnp.float32),
            jax.ShapeDtypeStruct((GRID_N, 1, 64), jnp.float32),
            jax.ShapeDtypeStruct((N, 1), jnp.float32),
        ],
    )(h64, att4, agg0, agg1, den0, den1, deg, b1r)


def _k2b_body(x1_ref, ps_ref, pss_ref, g_ref, b_ref, w2_ref, xw_out):
    S = jnp.sum(ps_ref[...], axis=0)
    SS = jnp.sum(pss_ref[...], axis=0)
    m = S / N
    v = SS / N - m * m
    sc = g_ref[...] * lax.rsqrt(v + 1e-5)
    sh = b_ref[...] - m * sc
    x1n = jnp.maximum(x1_ref[...] * sc + sh, 0.0)
    xw_out[...] = jnp.dot(x1n, w2_ref[...], preferred_element_type=jnp.float32)


def _k2b(x1, ps, pss, g1r, b1r, w2):
    return pl.pallas_call(
        _k2b_body,
        grid=(GRID_N,),
        in_specs=[
            pl.BlockSpec((BN, 64), lambda i: (i, 0)),
            pl.BlockSpec((GRID_N, 1, 64), lambda i: (0, 0, 0)),
            pl.BlockSpec((GRID_N, 1, 64), lambda i: (0, 0, 0)),
            pl.BlockSpec((1, 64), lambda i: (0, 0)),
            pl.BlockSpec((1, 64), lambda i: (0, 0)),
            pl.BlockSpec((64, 32), lambda i: (0, 0)),
        ],
        out_specs=pl.BlockSpec((BN, 32), lambda i: (i, 0)),
        out_shape=jax.ShapeDtypeStruct((N, 32), jnp.float32),
    )(x1, ps, pss, g1r, b1r, w2)


def _k3a_body(p0, p1, xw_ref, dinv_ref, b2_ref, x2_out, ps_out, pss_out):
    di = dinv_ref[...]
    x2 = p0[...] + p1[...] + xw_ref[...] * (di * di) + b2_ref[...]
    x2_out[...] = x2
    ps_out[...] = jnp.sum(x2, axis=0, keepdims=True)[None]
    pss_out[...] = jnp.sum(x2 * x2, axis=0, keepdims=True)[None]


def _k3a(p0, p1, xw, dinv, b2r):
    return pl.pallas_call(
        _k3a_body,
        grid=(GRID_N,),
        in_specs=[
            pl.BlockSpec((BN, 32), lambda i: (i, 0)),
            pl.BlockSpec((BN, 32), lambda i: (i, 0)),
            pl.BlockSpec((BN, 32), lambda i: (i, 0)),
            pl.BlockSpec((BN, 1), lambda i: (i, 0)),
            pl.BlockSpec((1, 32), lambda i: (0, 0)),
        ],
        out_specs=[
            pl.BlockSpec((BN, 32), lambda i: (i, 0)),
            pl.BlockSpec((1, 1, 32), lambda i: (i, 0, 0)),
            pl.BlockSpec((1, 1, 32), lambda i: (i, 0, 0)),
        ],
        out_shape=[
            jax.ShapeDtypeStruct((N, 32), jnp.float32),
            jax.ShapeDtypeStruct((GRID_N, 1, 32), jnp.float32),
            jax.ShapeDtypeStruct((GRID_N, 1, 32), jnp.float32),
        ],
    )(p0, p1, xw, dinv, b2r)


def _k3b_body(x2_ref, ps_ref, pss_ref, g_ref, b_ref, wf_ref, bf_ref, out_ref):
    S = jnp.sum(ps_ref[...], axis=0)
    SS = jnp.sum(pss_ref[...], axis=0)
    m = S / N
    v = SS / N - m * m
    sc = g_ref[...] * lax.rsqrt(v + 1e-5)
    sh = b_ref[...] - m * sc
    sc224 = jnp.concatenate([sc] * 7, axis=1)
    sh224 = jnp.concatenate([sh] * 7, axis=1)
    x2n = jnp.maximum(x2_ref[...] * sc224 + sh224, 0.0)
    out_ref[...] = (jnp.dot(x2n, wf_ref[...], preferred_element_type=jnp.float32)
                    + bf_ref[...])


def _k3b(x2r, ps, pss, g2r, b2r, wf, bfr):
    return pl.pallas_call(
        _k3b_body,
        grid=(5,),
        in_specs=[
            pl.BlockSpec((1000, 224), lambda i: (i, 0)),
            pl.BlockSpec((GRID_N, 1, 32), lambda i: (0, 0, 0)),
            pl.BlockSpec((GRID_N, 1, 32), lambda i: (0, 0, 0)),
            pl.BlockSpec((1, 32), lambda i: (0, 0)),
            pl.BlockSpec((1, 32), lambda i: (0, 0)),
            pl.BlockSpec((224, 8), lambda i: (0, 0)),
            pl.BlockSpec((1, 8), lambda i: (0, 0)),
        ],
        out_specs=pl.BlockSpec((1000, 8), lambda i: (i, 0)),
        out_shape=jax.ShapeDtypeStruct((5000, 8), jnp.float32),
    )(x2r, ps, pss, g2r, b2r, wf, bfr)


def kernel(x, edge_index, W1, a_src, a_dst, b1, gamma1, beta1,
           W2, b2, gamma2, beta2, Wf, bf):
    w1f = W1.reshape(128, 64)
    avs = a_src.reshape(1, 64)
    avd = a_dst.reshape(1, 64)

    h64, att4 = _k1(x, w1f, avs, avd)

    npad = NP - N
    h64p = jnp.pad(h64, ((0, npad), (0, 0)))
    h_st = jnp.concatenate([h64p[:, :32], h64p[:, 32:]], axis=0)
    att4p = jnp.pad(att4, ((0, npad), (0, 0)))
    att_sc = jnp.stack([jnp.stack([att4p[:, 0], att4p[:, 2]]),
                        jnp.stack([att4p[:, 1], att4p[:, 3]])])
    epad = EP - E
    src_pad = jnp.concatenate([edge_index[0], jnp.zeros((epad,), jnp.int32)])
    dst_pad = jnp.concatenate([edge_index[1], jnp.full((epad,), N, jnp.int32)])
    ei_pad = jnp.stack([src_pad, dst_pad])
    z48 = jnp.zeros((ROWS_PER_TILE, 48), jnp.float32)
    z32 = jnp.zeros((ROWS_PER_TILE, 32), jnp.float32)

    exh = _sc_att(ei_pad, att_sc)
    aggA = _sc_agg(ei_pad, h_st, exh, z48)

    agg0 = aggA[0, :N, :32]
    agg1 = aggA[1, :N, :32]
    den0 = aggA[0, :N, 32:33]
    den1 = aggA[1, :N, 32:33]
    deg = aggA[0, :N, 33:34]

    b1r = b1.reshape(1, 64)
    x1, ps1, pss1, dinv = _k2a(h64, att4, agg0, agg1, den0, den1, deg, b1r)
    xw = _k2b(x1, ps1, pss1, gamma1.reshape(1, 64), beta1.reshape(1, 64), W2)

    xw_pad = jnp.pad(xw, ((0, npad), (0, 0)))
    dinv_pad = jnp.pad(dinv[:, 0], (0, npad))

    outB = _sc_gcn(ei_pad, xw_pad, dinv_pad, z32)

    x2, ps2, pss2 = _k3a(outB[0, :N, :], outB[1, :N, :], xw, dinv,
                         b2.reshape(1, 32))
    out = _k3b(x2.reshape(5000, 224), ps2, pss2,
               gamma2.reshape(1, 32), beta2.reshape(1, 32), Wf,
               bf.reshape(1, 8))
    return out


# speedup vs baseline: 65.3038x; 1.0884x over previous
import functools

import jax
import jax.numpy as jnp
from jax import lax
from jax.experimental import pallas as pl
from jax.experimental.pallas import tpu as pltpu
from jax.experimental.pallas import tpu_sc as plsc

N = 35000
E = 560000
NP = 35072
EP = 573440
CHUNK = 128
ROWS_PER_TILE = NP // 16

BN = 1000
GRID_N = N // BN


def _k1_body(x_ref, w_ref, avs_ref, avd_ref, h_out, att_out):
    hb = jnp.dot(x_ref[...], w_ref[...], preferred_element_type=jnp.float32)
    h_out[0] = hb[:, :32]
    h_out[1] = hb[:, 32:]
    ts = hb * avs_ref[...]
    td = hb * avd_ref[...]
    as0 = jnp.sum(ts[:, :32], axis=1, keepdims=True)
    as1 = jnp.sum(ts[:, 32:], axis=1, keepdims=True)
    ad0 = jnp.sum(td[:, :32], axis=1, keepdims=True)
    ad1 = jnp.sum(td[:, 32:], axis=1, keepdims=True)
    att_out[...] = jnp.concatenate([as0, as1, ad0, ad1], axis=1)


def _k1(x, w1f, avs, avd):
    return pl.pallas_call(
        _k1_body,
        grid=(GRID_N,),
        in_specs=[
            pl.BlockSpec((BN, 128), lambda i: (i, 0)),
            pl.BlockSpec((128, 64), lambda i: (0, 0)),
            pl.BlockSpec((1, 64), lambda i: (0, 0)),
            pl.BlockSpec((1, 64), lambda i: (0, 0)),
        ],
        out_specs=[
            pl.BlockSpec((2, BN, 32), lambda i: (0, i, 0)),
            pl.BlockSpec((BN, 4), lambda i: (i, 0)),
        ],
        out_shape=[
            jax.ShapeDtypeStruct((2, NP, 32), jnp.float32),
            jax.ShapeDtypeStruct((N, 4), jnp.float32),
        ],
    )(x, w1f, avs, avd)


ACH = 1280

def _att_body(ei, att, ex_out, asrcv, adstv, idx, exb):
    c = lax.axis_index("c")
    s = lax.axis_index("s")
    pltpu.sync_copy(att.at[c, 0], asrcv)
    pltpu.sync_copy(att.at[c, 1], adstv)
    tile_base = s * (EP // 16)

    def body(i, carry):
        base = tile_base + i * ACH
        pltpu.sync_copy(ei.at[0, pl.ds(base, ACH)], idx.at[0])
        pltpu.sync_copy(ei.at[1, pl.ds(base, ACH)], idx.at[1])
        for g in range(ACH // 16):
            sv = idx[0, pl.ds(g * 16, 16)]
            dv = idx[1, pl.ds(g * 16, 16)]
            a_s = plsc.load_gather(asrcv, [sv])
            a_d = plsc.load_gather(adstv, [dv])
            e = a_s + a_d
            e = jnp.where(e > 0, e, 0.2 * e)
            exb[pl.ds(g * 16, 16)] = jnp.exp(e)
        pltpu.sync_copy(exb, ex_out.at[c, pl.ds(base, ACH)])
        return carry

    lax.fori_loop(0, EP // 16 // ACH, body, 0)


def _sc_att(ei_pad, att_sc):
    f = pl.kernel(
        _att_body,
        out_type=jax.ShapeDtypeStruct((2, EP), jnp.float32),
        mesh=plsc.VectorSubcoreMesh(core_axis_name="c", subcore_axis_name="s",
                                    num_cores=2, num_subcores=16),
        compiler_params=pltpu.CompilerParams(needs_layout_passes=False, use_tc_tiling_on_sc=False),
        scratch_types=[
            pltpu.VMEM((NP,), jnp.float32),
            pltpu.VMEM((NP,), jnp.float32),
            pltpu.VMEM((2, ACH), jnp.int32),
            pltpu.VMEM((ACH,), jnp.float32),
        ],
    )
    return f(ei_pad, att_sc)


def _agg_body(ei, h_st, exh, z32, z16, aggf_out, aggd_out,
              idx, gidx, rows, db0, db1, exv, accf, accd, gsem0, gsem1):
    c = lax.axis_index("c")
    s = lax.axis_index("s")
    pltpu.sync_copy(z32, accf.at[pl.ds(s * ROWS_PER_TILE, ROWS_PER_TILE)])
    pltpu.sync_copy(z16, accd.at[pl.ds(s * ROWS_PER_TILE, ROWS_PER_TILE)])
    plsc.subcore_barrier()

    tile_base = s * (EP // 16)
    iota16 = lax.iota(jnp.int32, 16)
    hoff = c * NP
    degval = jnp.where(c == 0, 1.0, 0.0).astype(jnp.float32)
    degrow = jnp.where(iota16 == 1, degval, 0.0).astype(jnp.float32)
    zcol = jnp.zeros((16,), jnp.int32)

    dbs = (db0, db1)
    for el in range(CHUNK):
        db0[el, :] = degrow
        db1[el, :] = degrow

    gsems = (gsem0, gsem1)

    def start_chunk(i, b):
        base = tile_base + i * CHUNK
        pltpu.sync_copy(ei.at[0, pl.ds(base, CHUNK)], idx.at[b, 0])
        pltpu.sync_copy(ei.at[1, pl.ds(base, CHUNK)], idx.at[b, 1])
        pltpu.sync_copy(exh.at[c, pl.ds(base, CHUNK)], exv.at[b])
        for g in range(8):
            sv = idx[b, 0, pl.ds(g * 16, 16)]
            gidx[b, pl.ds(g * 16, 16)] = sv + hoff
        pltpu.async_copy(h_st.at[gidx.at[b]], rows.at[b], gsems[b])

    def finish_chunk(b):
        pltpu.make_async_copy(h_st.at[gidx.at[b]], rows.at[b], gsems[b]).wait()
        for g in range(8):
            ex = exv[b, pl.ds(g * 16, 16)]
            plsc.store_scatter(dbs[b], [iota16 + g * 16, zcol], ex)
            for l in range(16):
                el = g * 16 + l
                exs = ex[l]
                rows[b, el, pl.ds(0, 16)] = rows[b, el, pl.ds(0, 16)] * exs
                rows[b, el, pl.ds(16, 16)] = rows[b, el, pl.ds(16, 16)] * exs
        pltpu.sync_copy(rows.at[b], accf.at[idx.at[b, 1]], add=True)
        pltpu.sync_copy(dbs[b], accd.at[idx.at[b, 1]], add=True)

    nch = EP // 16 // CHUNK
    start_chunk(0, 0)

    def body(j, carry):
        i0 = 2 * j
        start_chunk(i0 + 1, 1)
        finish_chunk(0)

        @pl.when(i0 + 2 < nch)
        def _():
            start_chunk(i0 + 2, 0)

        finish_chunk(1)
        return carry

    lax.fori_loop(0, nch // 2, body, 0)
    plsc.subcore_barrier()
    pltpu.sync_copy(accf.at[pl.ds(s * ROWS_PER_TILE, ROWS_PER_TILE)],
                    aggf_out.at[c, pl.ds(s * ROWS_PER_TILE, ROWS_PER_TILE)])
    pltpu.sync_copy(accd.at[pl.ds(s * ROWS_PER_TILE, ROWS_PER_TILE)],
                    aggd_out.at[c, pl.ds(s * ROWS_PER_TILE, ROWS_PER_TILE)])


def _sc_agg(ei_pad, h_st, exh, z32, z16):
    f = pl.kernel(
        _agg_body,
        out_type=[jax.ShapeDtypeStruct((2, NP, 32), jnp.float32),
                  jax.ShapeDtypeStruct((2, NP, 16), jnp.float32)],
        mesh=plsc.VectorSubcoreMesh(core_axis_name="c", subcore_axis_name="s",
                                    num_cores=2, num_subcores=16),
        compiler_params=pltpu.CompilerParams(needs_layout_passes=False, use_tc_tiling_on_sc=False),
        scratch_types=[
            pltpu.VMEM((2, 2, CHUNK), jnp.int32),
            pltpu.VMEM((2, CHUNK), jnp.int32),
            pltpu.VMEM((2, CHUNK, 32), jnp.float32),
            pltpu.VMEM((CHUNK, 16), jnp.float32),
            pltpu.VMEM((CHUNK, 16), jnp.float32),
            pltpu.VMEM((2, CHUNK), jnp.float32),
            pltpu.VMEM_SHARED((NP, 32), jnp.float32),
            pltpu.VMEM_SHARED((NP, 16), jnp.float32),
            pltpu.SemaphoreType.DMA,
            pltpu.SemaphoreType.DMA,
        ],
    )
    return f(ei_pad, h_st, exh, z32, z16)


def _gcn_body(ei, xw, dinv, z32, out,
              dv_tab, idx, rows, outb, acc, gsem0, gsem1):
    c = lax.axis_index("c")
    s = lax.axis_index("s")
    pltpu.sync_copy(z32, acc.at[pl.ds(s * ROWS_PER_TILE, ROWS_PER_TILE)])
    pltpu.sync_copy(dinv, dv_tab)
    plsc.subcore_barrier()

    tile_base = (c * 16 + s) * (EP // 32)
    gsems = (gsem0, gsem1)

    def start_chunk(i, b):
        base = tile_base + i * CHUNK
        pltpu.sync_copy(ei.at[0, pl.ds(base, CHUNK)], idx.at[b, 0])
        pltpu.sync_copy(ei.at[1, pl.ds(base, CHUNK)], idx.at[b, 1])
        pltpu.async_copy(xw.at[idx.at[b, 0]], rows.at[b], gsems[b])

    def finish_chunk(b):
        pltpu.make_async_copy(xw.at[idx.at[b, 0]], rows.at[b], gsems[b]).wait()
        for g in range(8):
            sv = idx[b, 0, pl.ds(g * 16, 16)]
            dv = idx[b, 1, pl.ds(g * 16, 16)]
            nv = plsc.load_gather(dv_tab, [sv]) * plsc.load_gather(dv_tab, [dv])
            for l in range(16):
                el = g * 16 + l
                ns = nv[l]
                outb[b, el, pl.ds(0, 16)] = rows[b, el, pl.ds(0, 16)] * ns
                outb[b, el, pl.ds(16, 16)] = rows[b, el, pl.ds(16, 16)] * ns
        pltpu.sync_copy(outb.at[b], acc.at[idx.at[b, 1]], add=True)

    nch = EP // 32 // CHUNK
    start_chunk(0, 0)

    def body(j, carry):
        i0 = 2 * j
        start_chunk(i0 + 1, 1)
        finish_chunk(0)

        @pl.when(i0 + 2 < nch)
        def _():
            start_chunk(i0 + 2, 0)

        finish_chunk(1)
        return carry

    lax.fori_loop(0, nch // 2, body, 0)
    plsc.subcore_barrier()
    pltpu.sync_copy(acc.at[pl.ds(s * ROWS_PER_TILE, ROWS_PER_TILE)],
                    out.at[c, pl.ds(s * ROWS_PER_TILE, ROWS_PER_TILE)])


def _sc_gcn(ei_pad, xw_pad, dinv_pad, z32):
    f = pl.kernel(
        _gcn_body,
        out_type=jax.ShapeDtypeStruct((2, NP, 32), jnp.float32),
        mesh=plsc.VectorSubcoreMesh(core_axis_name="c", subcore_axis_name="s",
                                    num_cores=2, num_subcores=16),
        compiler_params=pltpu.CompilerParams(needs_layout_passes=False, use_tc_tiling_on_sc=False),
        scratch_types=[
            pltpu.VMEM((NP,), jnp.float32),
            pltpu.VMEM((2, 2, CHUNK), jnp.int32),
            pltpu.VMEM((2, CHUNK, 32), jnp.float32),
            pltpu.VMEM((2, CHUNK, 32), jnp.float32),
            pltpu.VMEM_SHARED((NP, 32), jnp.float32),
            pltpu.SemaphoreType.DMA,
            pltpu.SemaphoreType.DMA,
        ],
    )
    return f(ei_pad, xw_pad, dinv_pad, z32)


def _k2a_body(h_ref, att_ref, aggf_ref, aggd_ref, b1_ref,
              x1_out, ps_out, pss_out, dinv_out):
    att = att_ref[...]

    def head(aggr, denr, asl, adl, hsl):
        es = asl + adl
        es = jnp.where(es > 0, es, 0.2 * es)
        exs = jnp.exp(es)
        den = denr + exs
        num = aggr + hsl * exs
        return num / (den + 1e-16)

    a0 = head(aggf_ref[0], aggd_ref[0, :, 0:1], att[:, 0:1], att[:, 2:3],
              h_ref[0])
    a1 = head(aggf_ref[1], aggd_ref[1, :, 0:1], att[:, 1:2], att[:, 3:4],
              h_ref[1])
    x1 = jnp.concatenate([a0, a1], axis=1) + b1_ref[...]
    x1_out[...] = x1
    ps_out[...] = jnp.sum(x1, axis=0, keepdims=True)[None]
    pss_out[...] = jnp.sum(x1 * x1, axis=0, keepdims=True)[None]
    dinv_out[...] = lax.rsqrt(aggd_ref[0, :, 1:2] + 1.0)


def _k2a(h2, att4, aggf, aggd, b1r):
    return pl.pallas_call(
        _k2a_body,
        grid=(GRID_N,),
        in_specs=[
            pl.BlockSpec((2, BN, 32), lambda i: (0, i, 0)),
            pl.BlockSpec((BN, 4), lambda i: (i, 0)),
            pl.BlockSpec((2, BN, 32), lambda i: (0, i, 0)),
            pl.BlockSpec((2, BN, 16), lambda i: (0, i, 0)),
            pl.BlockSpec((1, 64), lambda i: (0, 0)),
        ],
        out_specs=[
            pl.BlockSpec((BN, 64), lambda i: (i, 0)),
            pl.BlockSpec((1, 1, 64), lambda i: (i, 0, 0)),
            pl.BlockSpec((1, 1, 64), lambda i: (i, 0, 0)),
            pl.BlockSpec((BN, 1), lambda i: (i, 0)),
        ],
        out_shape=[
            jax.ShapeDtypeStruct((N, 64), jnp.float32),
            jax.ShapeDtypeStruct((GRID_N, 1, 64), jnp.float32),
            jax.ShapeDtypeStruct((GRID_N, 1, 64), jnp.float32),
            jax.ShapeDtypeStruct((NP, 1), jnp.float32),
        ],
    )(h2, att4, aggf, aggd, b1r)


def _k2b_body(x1_ref, ps_ref, pss_ref, g_ref, b_ref, w2_ref, xw_out):
    S = jnp.sum(ps_ref[...], axis=0)
    SS = jnp.sum(pss_ref[...], axis=0)
    m = S / N
    v = SS / N - m * m
    sc = g_ref[...] * lax.rsqrt(v + 1e-5)
    sh = b_ref[...] - m * sc
    x1n = jnp.maximum(x1_ref[...] * sc + sh, 0.0)
    xw_out[...] = jnp.dot(x1n, w2_ref[...], preferred_element_type=jnp.float32)


def _k2b(x1, ps, pss, g1r, b1r, w2):
    return pl.pallas_call(
        _k2b_body,
        grid=(GRID_N,),
        in_specs=[
            pl.BlockSpec((BN, 64), lambda i: (i, 0)),
            pl.BlockSpec((GRID_N, 1, 64), lambda i: (0, 0, 0)),
            pl.BlockSpec((GRID_N, 1, 64), lambda i: (0, 0, 0)),
            pl.BlockSpec((1, 64), lambda i: (0, 0)),
            pl.BlockSpec((1, 64), lambda i: (0, 0)),
            pl.BlockSpec((64, 32), lambda i: (0, 0)),
        ],
        out_specs=pl.BlockSpec((BN, 32), lambda i: (i, 0)),
        out_shape=jax.ShapeDtypeStruct((NP, 32), jnp.float32),
    )(x1, ps, pss, g1r, b1r, w2)


def _k3a_body(pb, xw_ref, dinv_ref, b2_ref, x2_out, ps_out, pss_out):
    di = dinv_ref[...]
    x2 = pb[0] + pb[1] + xw_ref[...] * (di * di) + b2_ref[...]
    x2_out[...] = x2
    ps_out[...] = jnp.sum(x2, axis=0, keepdims=True)[None]
    pss_out[...] = jnp.sum(x2 * x2, axis=0, keepdims=True)[None]


def _k3a(pb, xw, dinv, b2r):
    return pl.pallas_call(
        _k3a_body,
        grid=(GRID_N,),
        in_specs=[
            pl.BlockSpec((2, BN, 32), lambda i: (0, i, 0)),
            pl.BlockSpec((BN, 32), lambda i: (i, 0)),
            pl.BlockSpec((BN, 1), lambda i: (i, 0)),
            pl.BlockSpec((1, 32), lambda i: (0, 0)),
        ],
        out_specs=[
            pl.BlockSpec((BN, 32), lambda i: (i, 0)),
            pl.BlockSpec((1, 1, 32), lambda i: (i, 0, 0)),
            pl.BlockSpec((1, 1, 32), lambda i: (i, 0, 0)),
        ],
        out_shape=[
            jax.ShapeDtypeStruct((N, 32), jnp.float32),
            jax.ShapeDtypeStruct((GRID_N, 1, 32), jnp.float32),
            jax.ShapeDtypeStruct((GRID_N, 1, 32), jnp.float32),
        ],
    )(pb, xw, dinv, b2r)


def _k3b_body(x2_ref, ps_ref, pss_ref, g_ref, b_ref, wf_ref, bf_ref, out_ref):
    S = jnp.sum(ps_ref[...], axis=0)
    SS = jnp.sum(pss_ref[...], axis=0)
    m = S / N
    v = SS / N - m * m
    sc = g_ref[...] * lax.rsqrt(v + 1e-5)
    sh = b_ref[...] - m * sc
    sc224 = jnp.concatenate([sc] * 7, axis=1)
    sh224 = jnp.concatenate([sh] * 7, axis=1)
    x2n = jnp.maximum(x2_ref[...] * sc224 + sh224, 0.0)
    out_ref[...] = (jnp.dot(x2n, wf_ref[...], preferred_element_type=jnp.float32)
                    + bf_ref[...])


def _k3b(x2r, ps, pss, g2r, b2r, wf, bfr):
    return pl.pallas_call(
        _k3b_body,
        grid=(5,),
        in_specs=[
            pl.BlockSpec((1000, 224), lambda i: (i, 0)),
            pl.BlockSpec((GRID_N, 1, 32), lambda i: (0, 0, 0)),
            pl.BlockSpec((GRID_N, 1, 32), lambda i: (0, 0, 0)),
            pl.BlockSpec((1, 32), lambda i: (0, 0)),
            pl.BlockSpec((1, 32), lambda i: (0, 0)),
            pl.BlockSpec((224, 8), lambda i: (0, 0)),
            pl.BlockSpec((1, 8), lambda i: (0, 0)),
        ],
        out_specs=pl.BlockSpec((1000, 8), lambda i: (i, 0)),
        out_shape=jax.ShapeDtypeStruct((5000, 8), jnp.float32),
    )(x2r, ps, pss, g2r, b2r, wf, bfr)


def kernel(x, edge_index, W1, a_src, a_dst, b1, gamma1, beta1,
           W2, b2, gamma2, beta2, Wf, bf):
    w1f = W1.reshape(128, 64)
    avs = a_src.reshape(1, 64)
    avd = a_dst.reshape(1, 64)

    h2, att4 = _k1(x, w1f, avs, avd)

    npad = NP - N
    h_st = h2.reshape(2 * NP, 32)
    att4p = jnp.pad(att4, ((0, npad), (0, 0)))
    att_sc = jnp.stack([jnp.stack([att4p[:, 0], att4p[:, 2]]),
                        jnp.stack([att4p[:, 1], att4p[:, 3]])])
    epad = EP - E
    src_pad = jnp.concatenate([edge_index[0], jnp.zeros((epad,), jnp.int32)])
    dst_pad = jnp.concatenate([edge_index[1], jnp.full((epad,), N, jnp.int32)])
    ei_pad = jnp.stack([src_pad, dst_pad])
    z32 = jnp.zeros((ROWS_PER_TILE, 32), jnp.float32)
    z16 = jnp.zeros((ROWS_PER_TILE, 16), jnp.float32)

    exh = _sc_att(ei_pad, att_sc)
    aggF, aggD = _sc_agg(ei_pad, h_st, exh, z32, z16)

    b1r = b1.reshape(1, 64)
    x1, ps1, pss1, dinv = _k2a(h2, att4, aggF, aggD, b1r)
    xw = _k2b(x1, ps1, pss1, gamma1.reshape(1, 64), beta1.reshape(1, 64), W2)

    outB = _sc_gcn(ei_pad, xw, dinv.reshape(NP), z32)

    x2, ps2, pss2 = _k3a(outB, xw, dinv, b2.reshape(1, 32))
    out = _k3b(x2.reshape(5000, 224), ps2, pss2,
               gamma2.reshape(1, 32), beta2.reshape(1, 32), Wf,
               bf.reshape(1, 8))
    return out


# speedup vs baseline: 67.3683x; 1.0316x over previous
import functools

import jax
import jax.numpy as jnp
from jax import lax
from jax.experimental import pallas as pl
from jax.experimental.pallas import tpu as pltpu
from jax.experimental.pallas import tpu_sc as plsc

N = 35000
E = 560000
NP = 35072
EP = 573440
CHUNK = 128
ROWS_PER_TILE = NP // 16

BN = 1000
GRID_N = N // BN


def _k1_body(x_ref, w_ref, avs_ref, avd_ref, h_out, att_out):
    hb = jnp.dot(x_ref[...], w_ref[...], preferred_element_type=jnp.float32)
    h_out[0] = hb[:, :32]
    h_out[1] = hb[:, 32:]
    ts = hb * avs_ref[...]
    td = hb * avd_ref[...]
    as0 = jnp.sum(ts[:, :32], axis=1, keepdims=True)
    as1 = jnp.sum(ts[:, 32:], axis=1, keepdims=True)
    ad0 = jnp.sum(td[:, :32], axis=1, keepdims=True)
    ad1 = jnp.sum(td[:, 32:], axis=1, keepdims=True)
    att_out[...] = jnp.concatenate([as0, as1, ad0, ad1], axis=1)


def _k1(x, w1f, avs, avd):
    return pl.pallas_call(
        _k1_body,
        grid=(GRID_N,),
        in_specs=[
            pl.BlockSpec((BN, 128), lambda i: (i, 0)),
            pl.BlockSpec((128, 64), lambda i: (0, 0)),
            pl.BlockSpec((1, 64), lambda i: (0, 0)),
            pl.BlockSpec((1, 64), lambda i: (0, 0)),
        ],
        out_specs=[
            pl.BlockSpec((2, BN, 32), lambda i: (0, i, 0)),
            pl.BlockSpec((BN, 4), lambda i: (i, 0)),
        ],
        out_shape=[
            jax.ShapeDtypeStruct((2, NP, 32), jnp.float32),
            jax.ShapeDtypeStruct((N, 4), jnp.float32),
        ],
    )(x, w1f, avs, avd)


ACH = 1280

def _att_body(ei, att, ex_out, asrcv, adstv, idx, exb):
    c = lax.axis_index("c")
    s = lax.axis_index("s")
    pltpu.sync_copy(att.at[c, 0], asrcv)
    pltpu.sync_copy(att.at[c, 1], adstv)
    tile_base = s * (EP // 16)

    def body(i, carry):
        base = tile_base + i * ACH
        pltpu.sync_copy(ei.at[0, pl.ds(base, ACH)], idx.at[0])
        pltpu.sync_copy(ei.at[1, pl.ds(base, ACH)], idx.at[1])
        for g in range(ACH // 16):
            sv = idx[0, pl.ds(g * 16, 16)]
            dv = idx[1, pl.ds(g * 16, 16)]
            a_s = plsc.load_gather(asrcv, [sv])
            a_d = plsc.load_gather(adstv, [dv])
            e = a_s + a_d
            e = jnp.where(e > 0, e, 0.2 * e)
            exb[pl.ds(g * 16, 16)] = jnp.exp(e)
        pltpu.sync_copy(exb, ex_out.at[c, pl.ds(base, ACH)])
        return carry

    lax.fori_loop(0, EP // 16 // ACH, body, 0)


def _sc_att(ei_pad, att_sc):
    f = pl.kernel(
        _att_body,
        out_type=jax.ShapeDtypeStruct((2, EP), jnp.float32),
        mesh=plsc.VectorSubcoreMesh(core_axis_name="c", subcore_axis_name="s",
                                    num_cores=2, num_subcores=16),
        compiler_params=pltpu.CompilerParams(needs_layout_passes=False, use_tc_tiling_on_sc=False),
        scratch_types=[
            pltpu.VMEM((NP,), jnp.float32),
            pltpu.VMEM((NP,), jnp.float32),
            pltpu.VMEM((2, ACH), jnp.int32),
            pltpu.VMEM((ACH,), jnp.float32),
        ],
    )
    return f(ei_pad, att_sc)


def _agg_body(ei, h_st, exh, z48, agg_out,
              idx, gidx, rows, pay, exv, acc, gsem0, gsem1):
    c = lax.axis_index("c")
    s = lax.axis_index("s")
    pltpu.sync_copy(z48, acc.at[pl.ds(s * ROWS_PER_TILE, ROWS_PER_TILE)])
    plsc.subcore_barrier()

    tile_base = s * (EP // 16)
    iota16 = lax.iota(jnp.int32, 16)
    hoff = c * NP
    degval = jnp.where(c == 0, 1.0, 0.0).astype(jnp.float32)
    degrow = jnp.where(iota16 == 1, degval, 0.0).astype(jnp.float32)
    excol = jnp.full((16,), 32, jnp.int32)

    for el in range(CHUNK):
        pay[el, pl.ds(32, 16)] = degrow

    gsems = (gsem0, gsem1)

    def start_chunk(i, b):
        base = tile_base + i * CHUNK
        pltpu.sync_copy(ei.at[0, pl.ds(base, CHUNK)], idx.at[b, 0])
        pltpu.sync_copy(ei.at[1, pl.ds(base, CHUNK)], idx.at[b, 1])
        pltpu.sync_copy(exh.at[c, pl.ds(base, CHUNK)], exv.at[b])
        for g in range(8):
            sv = idx[b, 0, pl.ds(g * 16, 16)]
            gidx[b, pl.ds(g * 16, 16)] = sv + hoff
        pltpu.async_copy(h_st.at[gidx.at[b]], rows.at[b], gsems[b])

    def finish_chunk(b):
        pltpu.make_async_copy(h_st.at[gidx.at[b]], rows.at[b], gsems[b]).wait()
        for g in range(8):
            ex = exv[b, pl.ds(g * 16, 16)]
            plsc.store_scatter(pay, [iota16 + g * 16, excol], ex)
            for l in range(16):
                el = g * 16 + l
                exs = ex[l]
                pay[el, pl.ds(0, 16)] = rows[b, el, pl.ds(0, 16)] * exs
                pay[el, pl.ds(16, 16)] = rows[b, el, pl.ds(16, 16)] * exs
        pltpu.sync_copy(pay, acc.at[idx.at[b, 1]], add=True)

    nch = EP // 16 // CHUNK
    start_chunk(0, 0)

    def body(j, carry):
        i0 = 2 * j
        start_chunk(i0 + 1, 1)
        finish_chunk(0)

        @pl.when(i0 + 2 < nch)
        def _():
            start_chunk(i0 + 2, 0)

        finish_chunk(1)
        return carry

    lax.fori_loop(0, nch // 2, body, 0)
    plsc.subcore_barrier()
    pltpu.sync_copy(acc.at[pl.ds(s * ROWS_PER_TILE, ROWS_PER_TILE)],
                    agg_out.at[c, pl.ds(s * ROWS_PER_TILE, ROWS_PER_TILE)])


def _sc_agg(ei_pad, h_st, exh, z48):
    f = pl.kernel(
        _agg_body,
        out_type=jax.ShapeDtypeStruct((2, NP, 48), jnp.float32),
        mesh=plsc.VectorSubcoreMesh(core_axis_name="c", subcore_axis_name="s",
                                    num_cores=2, num_subcores=16),
        compiler_params=pltpu.CompilerParams(needs_layout_passes=False, use_tc_tiling_on_sc=False),
        scratch_types=[
            pltpu.VMEM((2, 2, CHUNK), jnp.int32),
            pltpu.VMEM((2, CHUNK), jnp.int32),
            pltpu.VMEM((2, CHUNK, 32), jnp.float32),
            pltpu.VMEM((CHUNK, 48), jnp.float32),
            pltpu.VMEM((2, CHUNK), jnp.float32),
            pltpu.VMEM_SHARED((NP, 48), jnp.float32),
            pltpu.SemaphoreType.DMA,
            pltpu.SemaphoreType.DMA,
        ],
    )
    return f(ei_pad, h_st, exh, z48)


def _gcn_body(ei, xw, dinv, z32, out,
              dv_tab, idx, rows, outb, acc, gsem0, gsem1):
    c = lax.axis_index("c")
    s = lax.axis_index("s")
    pltpu.sync_copy(z32, acc.at[pl.ds(s * ROWS_PER_TILE, ROWS_PER_TILE)])
    pltpu.sync_copy(dinv, dv_tab)
    plsc.subcore_barrier()

    tile_base = (c * 16 + s) * (EP // 32)
    gsems = (gsem0, gsem1)

    def start_chunk(i, b):
        base = tile_base + i * CHUNK
        pltpu.sync_copy(ei.at[0, pl.ds(base, CHUNK)], idx.at[b, 0])
        pltpu.sync_copy(ei.at[1, pl.ds(base, CHUNK)], idx.at[b, 1])
        pltpu.async_copy(xw.at[idx.at[b, 0]], rows.at[b], gsems[b])

    def finish_chunk(b):
        pltpu.make_async_copy(xw.at[idx.at[b, 0]], rows.at[b], gsems[b]).wait()
        for g in range(8):
            sv = idx[b, 0, pl.ds(g * 16, 16)]
            dv = idx[b, 1, pl.ds(g * 16, 16)]
            nv = plsc.load_gather(dv_tab, [sv]) * plsc.load_gather(dv_tab, [dv])
            for l in range(16):
                el = g * 16 + l
                ns = nv[l]
                outb[b, el, pl.ds(0, 16)] = rows[b, el, pl.ds(0, 16)] * ns
                outb[b, el, pl.ds(16, 16)] = rows[b, el, pl.ds(16, 16)] * ns
        pltpu.sync_copy(outb.at[b], acc.at[idx.at[b, 1]], add=True)

    nch = EP // 32 // CHUNK
    start_chunk(0, 0)

    def body(j, carry):
        i0 = 2 * j
        start_chunk(i0 + 1, 1)
        finish_chunk(0)

        @pl.when(i0 + 2 < nch)
        def _():
            start_chunk(i0 + 2, 0)

        finish_chunk(1)
        return carry

    lax.fori_loop(0, nch // 2, body, 0)
    plsc.subcore_barrier()
    pltpu.sync_copy(acc.at[pl.ds(s * ROWS_PER_TILE, ROWS_PER_TILE)],
                    out.at[c, pl.ds(s * ROWS_PER_TILE, ROWS_PER_TILE)])


def _sc_gcn(ei_pad, xw_pad, dinv_pad, z32):
    f = pl.kernel(
        _gcn_body,
        out_type=jax.ShapeDtypeStruct((2, NP, 32), jnp.float32),
        mesh=plsc.VectorSubcoreMesh(core_axis_name="c", subcore_axis_name="s",
                                    num_cores=2, num_subcores=16),
        compiler_params=pltpu.CompilerParams(needs_layout_passes=False, use_tc_tiling_on_sc=False),
        scratch_types=[
            pltpu.VMEM((NP,), jnp.float32),
            pltpu.VMEM((2, 2, CHUNK), jnp.int32),
            pltpu.VMEM((2, CHUNK, 32), jnp.float32),
            pltpu.VMEM((2, CHUNK, 32), jnp.float32),
            pltpu.VMEM_SHARED((NP, 32), jnp.float32),
            pltpu.SemaphoreType.DMA,
            pltpu.SemaphoreType.DMA,
        ],
    )
    return f(ei_pad, xw_pad, dinv_pad, z32)


def _k2a_body(h_ref, att_ref, agg_ref, b1_ref,
              x1_out, ps_out, pss_out, dinv_out):
    att = att_ref[...]

    def head(aggr, denr, asl, adl, hsl):
        es = asl + adl
        es = jnp.where(es > 0, es, 0.2 * es)
        exs = jnp.exp(es)
        den = denr + exs
        num = aggr + hsl * exs
        return num / (den + 1e-16)

    a0 = head(agg_ref[0, :, 0:32], agg_ref[0, :, 32:33], att[:, 0:1],
              att[:, 2:3], h_ref[0])
    a1 = head(agg_ref[1, :, 0:32], agg_ref[1, :, 32:33], att[:, 1:2],
              att[:, 3:4], h_ref[1])
    x1 = jnp.concatenate([a0, a1], axis=1) + b1_ref[...]
    x1_out[...] = x1
    ps_out[...] = jnp.sum(x1, axis=0, keepdims=True)[None]
    pss_out[...] = jnp.sum(x1 * x1, axis=0, keepdims=True)[None]
    dinv_out[...] = lax.rsqrt(agg_ref[0, :, 33:34] + 1.0)


def _k2a(h2, att4, agg48, b1r):
    return pl.pallas_call(
        _k2a_body,
        grid=(GRID_N,),
        in_specs=[
            pl.BlockSpec((2, BN, 32), lambda i: (0, i, 0)),
            pl.BlockSpec((BN, 4), lambda i: (i, 0)),
            pl.BlockSpec((2, BN, 48), lambda i: (0, i, 0)),
            pl.BlockSpec((1, 64), lambda i: (0, 0)),
        ],
        out_specs=[
            pl.BlockSpec((BN, 64), lambda i: (i, 0)),
            pl.BlockSpec((1, 1, 64), lambda i: (i, 0, 0)),
            pl.BlockSpec((1, 1, 64), lambda i: (i, 0, 0)),
            pl.BlockSpec((BN, 1), lambda i: (i, 0)),
        ],
        out_shape=[
            jax.ShapeDtypeStruct((N, 64), jnp.float32),
            jax.ShapeDtypeStruct((GRID_N, 1, 64), jnp.float32),
            jax.ShapeDtypeStruct((GRID_N, 1, 64), jnp.float32),
            jax.ShapeDtypeStruct((NP, 1), jnp.float32),
        ],
    )(h2, att4, agg48, b1r)


def _k2b_body(x1_ref, ps_ref, pss_ref, g_ref, b_ref, w2_ref, xw_out):
    S = jnp.sum(ps_ref[...], axis=0)
    SS = jnp.sum(pss_ref[...], axis=0)
    m = S / N
    v = SS / N - m * m
    sc = g_ref[...] * lax.rsqrt(v + 1e-5)
    sh = b_ref[...] - m * sc
    x1n = jnp.maximum(x1_ref[...] * sc + sh, 0.0)
    xw_out[...] = jnp.dot(x1n, w2_ref[...], preferred_element_type=jnp.float32)


def _k2b(x1, ps, pss, g1r, b1r, w2):
    return pl.pallas_call(
        _k2b_body,
        grid=(GRID_N,),
        in_specs=[
            pl.BlockSpec((BN, 64), lambda i: (i, 0)),
            pl.BlockSpec((GRID_N, 1, 64), lambda i: (0, 0, 0)),
            pl.BlockSpec((GRID_N, 1, 64), lambda i: (0, 0, 0)),
            pl.BlockSpec((1, 64), lambda i: (0, 0)),
            pl.BlockSpec((1, 64), lambda i: (0, 0)),
            pl.BlockSpec((64, 32), lambda i: (0, 0)),
        ],
        out_specs=pl.BlockSpec((BN, 32), lambda i: (i, 0)),
        out_shape=jax.ShapeDtypeStruct((NP, 32), jnp.float32),
    )(x1, ps, pss, g1r, b1r, w2)


def _k3a_body(pb, xw_ref, dinv_ref, b2_ref, x2_out, ps_out, pss_out):
    di = dinv_ref[...]
    x2 = pb[0] + pb[1] + xw_ref[...] * (di * di) + b2_ref[...]
    x2_out[...] = x2
    ps_out[...] = jnp.sum(x2, axis=0, keepdims=True)[None]
    pss_out[...] = jnp.sum(x2 * x2, axis=0, keepdims=True)[None]


def _k3a(pb, xw, dinv, b2r):
    return pl.pallas_call(
        _k3a_body,
        grid=(GRID_N,),
        in_specs=[
            pl.BlockSpec((2, BN, 32), lambda i: (0, i, 0)),
            pl.BlockSpec((BN, 32), lambda i: (i, 0)),
            pl.BlockSpec((BN, 1), lambda i: (i, 0)),
            pl.BlockSpec((1, 32), lambda i: (0, 0)),
        ],
        out_specs=[
            pl.BlockSpec((BN, 32), lambda i: (i, 0)),
            pl.BlockSpec((1, 1, 32), lambda i: (i, 0, 0)),
            pl.BlockSpec((1, 1, 32), lambda i: (i, 0, 0)),
        ],
        out_shape=[
            jax.ShapeDtypeStruct((N, 32), jnp.float32),
            jax.ShapeDtypeStruct((GRID_N, 1, 32), jnp.float32),
            jax.ShapeDtypeStruct((GRID_N, 1, 32), jnp.float32),
        ],
    )(pb, xw, dinv, b2r)


def _k3b_body(x2_ref, ps_ref, pss_ref, g_ref, b_ref, wf_ref, bf_ref, out_ref):
    S = jnp.sum(ps_ref[...], axis=0)
    SS = jnp.sum(pss_ref[...], axis=0)
    m = S / N
    v = SS / N - m * m
    sc = g_ref[...] * lax.rsqrt(v + 1e-5)
    sh = b_ref[...] - m * sc
    sc224 = jnp.concatenate([sc] * 7, axis=1)
    sh224 = jnp.concatenate([sh] * 7, axis=1)
    x2n = jnp.maximum(x2_ref[...] * sc224 + sh224, 0.0)
    out_ref[...] = (jnp.dot(x2n, wf_ref[...], preferred_element_type=jnp.float32)
                    + bf_ref[...])


def _k3b(x2r, ps, pss, g2r, b2r, wf, bfr):
    return pl.pallas_call(
        _k3b_body,
        grid=(5,),
        in_specs=[
            pl.BlockSpec((1000, 224), lambda i: (i, 0)),
            pl.BlockSpec((GRID_N, 1, 32), lambda i: (0, 0, 0)),
            pl.BlockSpec((GRID_N, 1, 32), lambda i: (0, 0, 0)),
            pl.BlockSpec((1, 32), lambda i: (0, 0)),
            pl.BlockSpec((1, 32), lambda i: (0, 0)),
            pl.BlockSpec((224, 8), lambda i: (0, 0)),
            pl.BlockSpec((1, 8), lambda i: (0, 0)),
        ],
        out_specs=pl.BlockSpec((1000, 8), lambda i: (i, 0)),
        out_shape=jax.ShapeDtypeStruct((5000, 8), jnp.float32),
    )(x2r, ps, pss, g2r, b2r, wf, bfr)


def kernel(x, edge_index, W1, a_src, a_dst, b1, gamma1, beta1,
           W2, b2, gamma2, beta2, Wf, bf):
    w1f = W1.reshape(128, 64)
    avs = a_src.reshape(1, 64)
    avd = a_dst.reshape(1, 64)

    h2, att4 = _k1(x, w1f, avs, avd)

    npad = NP - N
    h_st = h2.reshape(2 * NP, 32)
    att4p = jnp.pad(att4, ((0, npad), (0, 0)))
    att_sc = jnp.stack([jnp.stack([att4p[:, 0], att4p[:, 2]]),
                        jnp.stack([att4p[:, 1], att4p[:, 3]])])
    epad = EP - E
    src_pad = jnp.concatenate([edge_index[0], jnp.zeros((epad,), jnp.int32)])
    dst_pad = jnp.concatenate([edge_index[1], jnp.full((epad,), N, jnp.int32)])
    ei_pad = jnp.stack([src_pad, dst_pad])
    z32 = jnp.zeros((ROWS_PER_TILE, 32), jnp.float32)
    z48 = jnp.zeros((ROWS_PER_TILE, 48), jnp.float32)

    exh = _sc_att(ei_pad, att_sc)
    agg48 = _sc_agg(ei_pad, h_st, exh, z48)

    b1r = b1.reshape(1, 64)
    x1, ps1, pss1, dinv = _k2a(h2, att4, agg48, b1r)
    xw = _k2b(x1, ps1, pss1, gamma1.reshape(1, 64), beta1.reshape(1, 64), W2)

    outB = _sc_gcn(ei_pad, xw, dinv.reshape(NP), z32)

    x2, ps2, pss2 = _k3a(outB, xw, dinv, b2.reshape(1, 32))
    out = _k3b(x2.reshape(5000, 224), ps2, pss2,
               gamma2.reshape(1, 32), beta2.reshape(1, 32), Wf,
               bf.reshape(1, 8))
    return out


# speedup vs baseline: 80.6253x; 1.1968x over previous
import functools

import jax
import jax.numpy as jnp
from jax import lax
from jax.experimental import pallas as pl
from jax.experimental.pallas import tpu as pltpu
from jax.experimental.pallas import tpu_sc as plsc

N = 35000
E = 560000
NP = 35072
EP = 573440
CHUNK = 128
ROWS_PER_TILE = NP // 16

BN = 1000
GRID_N = N // BN


def _k1_body(x_ref, w_ref, avs_ref, avd_ref, h_out, att_out):
    hb = jnp.dot(x_ref[...], w_ref[...], preferred_element_type=jnp.float32)
    h_out[0] = hb[:, :32]
    h_out[1] = hb[:, 32:]
    ts = hb * avs_ref[...]
    td = hb * avd_ref[...]
    as0 = jnp.sum(ts[:, :32], axis=1, keepdims=True)
    as1 = jnp.sum(ts[:, 32:], axis=1, keepdims=True)
    ad0 = jnp.sum(td[:, :32], axis=1, keepdims=True)
    ad1 = jnp.sum(td[:, 32:], axis=1, keepdims=True)
    att_out[...] = jnp.concatenate([as0, as1, ad0, ad1], axis=1)


def _k1(x, w1f, avs, avd):
    return pl.pallas_call(
        _k1_body,
        grid=(GRID_N,),
        in_specs=[
            pl.BlockSpec((BN, 128), lambda i: (i, 0)),
            pl.BlockSpec((128, 64), lambda i: (0, 0)),
            pl.BlockSpec((1, 64), lambda i: (0, 0)),
            pl.BlockSpec((1, 64), lambda i: (0, 0)),
        ],
        out_specs=[
            pl.BlockSpec((2, BN, 32), lambda i: (0, i, 0)),
            pl.BlockSpec((BN, 4), lambda i: (i, 0)),
        ],
        out_shape=[
            jax.ShapeDtypeStruct((2, NP, 32), jnp.float32),
            jax.ShapeDtypeStruct((N, 4), jnp.float32),
        ],
    )(x, w1f, avs, avd)


ACH = 1280

def _att_body(ei, att, ex_out, asrcv, adstv, idx, exb):
    c = lax.axis_index("c")
    s = lax.axis_index("s")
    pltpu.sync_copy(att.at[c, 0], asrcv)
    pltpu.sync_copy(att.at[c, 1], adstv)
    tile_base = s * (EP // 16)

    def body(i, carry):
        base = tile_base + i * ACH
        pltpu.sync_copy(ei.at[0, pl.ds(base, ACH)], idx.at[0])
        pltpu.sync_copy(ei.at[1, pl.ds(base, ACH)], idx.at[1])
        for g in range(ACH // 16):
            sv = idx[0, pl.ds(g * 16, 16)]
            dv = idx[1, pl.ds(g * 16, 16)]
            a_s = plsc.load_gather(asrcv, [sv])
            a_d = plsc.load_gather(adstv, [dv])
            e = a_s + a_d
            e = jnp.where(e > 0, e, 0.2 * e)
            exb[pl.ds(g * 16, 16)] = jnp.exp(e)
        pltpu.sync_copy(exb, ex_out.at[c, pl.ds(base, ACH)])
        return carry

    lax.fori_loop(0, EP // 16 // ACH, body, 0)


def _sc_att(ei_pad, att_sc):
    f = pl.kernel(
        _att_body,
        out_type=jax.ShapeDtypeStruct((2, EP), jnp.float32),
        mesh=plsc.VectorSubcoreMesh(core_axis_name="c", subcore_axis_name="s",
                                    num_cores=2, num_subcores=16),
        compiler_params=pltpu.CompilerParams(needs_layout_passes=False, use_tc_tiling_on_sc=False),
        scratch_types=[
            pltpu.VMEM((NP,), jnp.float32),
            pltpu.VMEM((NP,), jnp.float32),
            pltpu.VMEM((2, ACH), jnp.int32),
            pltpu.VMEM((ACH,), jnp.float32),
        ],
    )
    return f(ei_pad, att_sc)


def _agg_body(ei, h_st, exh, z48, agg_out,
              idx, exv, gidx, rows, pay, acc,
              psem0, psem1, psem2, psem3, gsem0, gsem1):
    c = lax.axis_index("c")
    s = lax.axis_index("s")
    pltpu.sync_copy(z48, acc.at[pl.ds(s * ROWS_PER_TILE, ROWS_PER_TILE)])
    plsc.subcore_barrier()

    tile_base = s * (EP // 16)
    iota16 = lax.iota(jnp.int32, 16)
    hoff = c * NP
    degval = jnp.where(c == 0, 1.0, 0.0).astype(jnp.float32)
    degrow = jnp.where(iota16 == 1, degval, 0.0).astype(jnp.float32)
    excol = jnp.full((16,), 32, jnp.int32)

    for el in range(CHUNK):
        pay[el, pl.ds(32, 16)] = degrow

    psems = (psem0, psem1, psem2, psem3)
    gsems = (gsem0, gsem1)
    nch = EP // 16 // CHUNK

    def prefetch(i, b):
        base = tile_base + i * CHUNK
        pltpu.async_copy(ei.at[0, pl.ds(base, CHUNK)], idx.at[b, 0], psems[b])
        pltpu.async_copy(ei.at[1, pl.ds(base, CHUNK)], idx.at[b, 1], psems[b])
        pltpu.async_copy(exh.at[c, pl.ds(base, CHUNK)], exv.at[b], psems[b])

    def start_gather(i, b, g):
        base = tile_base + i * CHUNK
        pltpu.make_async_copy(ei.at[0, pl.ds(base, CHUNK)], idx.at[b, 0],
                              psems[b]).wait()
        pltpu.make_async_copy(ei.at[1, pl.ds(base, CHUNK)], idx.at[b, 1],
                              psems[b]).wait()
        pltpu.make_async_copy(exh.at[c, pl.ds(base, CHUNK)], exv.at[b],
                              psems[b]).wait()
        for gg in range(8):
            sv = idx[b, 0, pl.ds(gg * 16, 16)]
            gidx[g, pl.ds(gg * 16, 16)] = sv + hoff
        pltpu.async_copy(h_st.at[gidx.at[g]], rows.at[g], gsems[g])

    def finish(b, g):
        pltpu.make_async_copy(h_st.at[gidx.at[g]], rows.at[g], gsems[g]).wait()
        for gg in range(8):
            ex = exv[b, pl.ds(gg * 16, 16)]
            plsc.store_scatter(pay, [iota16 + gg * 16, excol], ex)
            for l in range(16):
                el = gg * 16 + l
                exs = ex[l]
                pay[el, pl.ds(0, 16)] = rows[g, el, pl.ds(0, 16)] * exs
                pay[el, pl.ds(16, 16)] = rows[g, el, pl.ds(16, 16)] * exs
        pltpu.sync_copy(pay, acc.at[idx.at[b, 1]], add=True)

    for b in range(4):
        prefetch(b, b)
    start_gather(0, 0, 0)

    def body(j, carry):
        i0 = 4 * j
        start_gather(i0 + 1, 1, 1)
        finish(0, 0)

        @pl.when(i0 + 4 < nch)
        def _():
            prefetch(i0 + 4, 0)

        start_gather(i0 + 2, 2, 0)
        finish(1, 1)

        @pl.when(i0 + 5 < nch)
        def _():
            prefetch(i0 + 5, 1)

        start_gather(i0 + 3, 3, 1)
        finish(2, 0)

        @pl.when(i0 + 6 < nch)
        def _():
            prefetch(i0 + 6, 2)

        @pl.when(i0 + 4 < nch)
        def _():
            start_gather(i0 + 4, 0, 0)

        finish(3, 1)

        @pl.when(i0 + 7 < nch)
        def _():
            prefetch(i0 + 7, 3)

        return carry

    lax.fori_loop(0, nch // 4, body, 0)
    plsc.subcore_barrier()
    pltpu.sync_copy(acc.at[pl.ds(s * ROWS_PER_TILE, ROWS_PER_TILE)],
                    agg_out.at[c, pl.ds(s * ROWS_PER_TILE, ROWS_PER_TILE)])


def _sc_agg(ei_pad, h_st, exh, z48):
    f = pl.kernel(
        _agg_body,
        out_type=jax.ShapeDtypeStruct((2, NP, 48), jnp.float32),
        mesh=plsc.VectorSubcoreMesh(core_axis_name="c", subcore_axis_name="s",
                                    num_cores=2, num_subcores=16),
        compiler_params=pltpu.CompilerParams(needs_layout_passes=False, use_tc_tiling_on_sc=False),
        scratch_types=[
            pltpu.VMEM((4, 2, CHUNK), jnp.int32),
            pltpu.VMEM((4, CHUNK), jnp.float32),
            pltpu.VMEM((2, CHUNK), jnp.int32),
            pltpu.VMEM((2, CHUNK, 32), jnp.float32),
            pltpu.VMEM((CHUNK, 48), jnp.float32),
            pltpu.VMEM_SHARED((NP, 48), jnp.float32),
            pltpu.SemaphoreType.DMA,
            pltpu.SemaphoreType.DMA,
            pltpu.SemaphoreType.DMA,
            pltpu.SemaphoreType.DMA,
            pltpu.SemaphoreType.DMA,
            pltpu.SemaphoreType.DMA,
        ],
    )
    return f(ei_pad, h_st, exh, z48)


def _gcn_body(ei, xw, dinv, z32, out,
              dv_tab, idx, rows, outb, acc, gsem0, gsem1):
    c = lax.axis_index("c")
    s = lax.axis_index("s")
    pltpu.sync_copy(z32, acc.at[pl.ds(s * ROWS_PER_TILE, ROWS_PER_TILE)])
    pltpu.sync_copy(dinv, dv_tab)
    plsc.subcore_barrier()

    tile_base = (c * 16 + s) * (EP // 32)
    gsems = (gsem0, gsem1)

    def start_chunk(i, b):
        base = tile_base + i * CHUNK
        pltpu.sync_copy(ei.at[0, pl.ds(base, CHUNK)], idx.at[b, 0])
        pltpu.sync_copy(ei.at[1, pl.ds(base, CHUNK)], idx.at[b, 1])
        pltpu.async_copy(xw.at[idx.at[b, 0]], rows.at[b], gsems[b])

    def finish_chunk(b):
        pltpu.make_async_copy(xw.at[idx.at[b, 0]], rows.at[b], gsems[b]).wait()
        for g in range(8):
            sv = idx[b, 0, pl.ds(g * 16, 16)]
            dv = idx[b, 1, pl.ds(g * 16, 16)]
            nv = plsc.load_gather(dv_tab, [sv]) * plsc.load_gather(dv_tab, [dv])
            for l in range(16):
                el = g * 16 + l
                ns = nv[l]
                outb[b, el, pl.ds(0, 16)] = rows[b, el, pl.ds(0, 16)] * ns
                outb[b, el, pl.ds(16, 16)] = rows[b, el, pl.ds(16, 16)] * ns
        pltpu.sync_copy(outb.at[b], acc.at[idx.at[b, 1]], add=True)

    nch = EP // 32 // CHUNK
    start_chunk(0, 0)

    def body(j, carry):
        i0 = 2 * j
        start_chunk(i0 + 1, 1)
        finish_chunk(0)

        @pl.when(i0 + 2 < nch)
        def _():
            start_chunk(i0 + 2, 0)

        finish_chunk(1)
        return carry

    lax.fori_loop(0, nch // 2, body, 0)
    plsc.subcore_barrier()
    pltpu.sync_copy(acc.at[pl.ds(s * ROWS_PER_TILE, ROWS_PER_TILE)],
                    out.at[c, pl.ds(s * ROWS_PER_TILE, ROWS_PER_TILE)])


def _sc_gcn(ei_pad, xw_pad, dinv_pad, z32):
    f = pl.kernel(
        _gcn_body,
        out_type=jax.ShapeDtypeStruct((2, NP, 32), jnp.float32),
        mesh=plsc.VectorSubcoreMesh(core_axis_name="c", subcore_axis_name="s",
                                    num_cores=2, num_subcores=16),
        compiler_params=pltpu.CompilerParams(needs_layout_passes=False, use_tc_tiling_on_sc=False),
        scratch_types=[
            pltpu.VMEM((NP,), jnp.float32),
            pltpu.VMEM((2, 2, CHUNK), jnp.int32),
            pltpu.VMEM((2, CHUNK, 32), jnp.float32),
            pltpu.VMEM((2, CHUNK, 32), jnp.float32),
            pltpu.VMEM_SHARED((NP, 32), jnp.float32),
            pltpu.SemaphoreType.DMA,
            pltpu.SemaphoreType.DMA,
        ],
    )
    return f(ei_pad, xw_pad, dinv_pad, z32)


def _k2a_body(h_ref, att_ref, agg_ref, b1_ref,
              x1_out, ps_out, pss_out, dinv_out):
    att = att_ref[...]

    def head(aggr, denr, asl, adl, hsl):
        es = asl + adl
        es = jnp.where(es > 0, es, 0.2 * es)
        exs = jnp.exp(es)
        den = denr + exs
        num = aggr + hsl * exs
        return num / (den + 1e-16)

    a0 = head(agg_ref[0, :, 0:32], agg_ref[0, :, 32:33], att[:, 0:1],
              att[:, 2:3], h_ref[0])
    a1 = head(agg_ref[1, :, 0:32], agg_ref[1, :, 32:33], att[:, 1:2],
              att[:, 3:4], h_ref[1])
    x1 = jnp.concatenate([a0, a1], axis=1) + b1_ref[...]
    x1_out[...] = x1
    ps_out[...] = jnp.sum(x1, axis=0, keepdims=True)[None]
    pss_out[...] = jnp.sum(x1 * x1, axis=0, keepdims=True)[None]
    dinv_out[...] = lax.rsqrt(agg_ref[0, :, 33:34] + 1.0)


def _k2a(h2, att4, agg48, b1r):
    return pl.pallas_call(
        _k2a_body,
        grid=(GRID_N,),
        in_specs=[
            pl.BlockSpec((2, BN, 32), lambda i: (0, i, 0)),
            pl.BlockSpec((BN, 4), lambda i: (i, 0)),
            pl.BlockSpec((2, BN, 48), lambda i: (0, i, 0)),
            pl.BlockSpec((1, 64), lambda i: (0, 0)),
        ],
        out_specs=[
            pl.BlockSpec((BN, 64), lambda i: (i, 0)),
            pl.BlockSpec((1, 1, 64), lambda i: (i, 0, 0)),
            pl.BlockSpec((1, 1, 64), lambda i: (i, 0, 0)),
            pl.BlockSpec((BN, 1), lambda i: (i, 0)),
        ],
        out_shape=[
            jax.ShapeDtypeStruct((N, 64), jnp.float32),
            jax.ShapeDtypeStruct((GRID_N, 1, 64), jnp.float32),
            jax.ShapeDtypeStruct((GRID_N, 1, 64), jnp.float32),
            jax.ShapeDtypeStruct((NP, 1), jnp.float32),
        ],
    )(h2, att4, agg48, b1r)


def _k2b_body(x1_ref, ps_ref, pss_ref, g_ref, b_ref, w2_ref, xw_out):
    S = jnp.sum(ps_ref[...], axis=0)
    SS = jnp.sum(pss_ref[...], axis=0)
    m = S / N
    v = SS / N - m * m
    sc = g_ref[...] * lax.rsqrt(v + 1e-5)
    sh = b_ref[...] - m * sc
    x1n = jnp.maximum(x1_ref[...] * sc + sh, 0.0)
    xw_out[...] = jnp.dot(x1n, w2_ref[...], preferred_element_type=jnp.float32)


def _k2b(x1, ps, pss, g1r, b1r, w2):
    return pl.pallas_call(
        _k2b_body,
        grid=(GRID_N,),
        in_specs=[
            pl.BlockSpec((BN, 64), lambda i: (i, 0)),
            pl.BlockSpec((GRID_N, 1, 64), lambda i: (0, 0, 0)),
            pl.BlockSpec((GRID_N, 1, 64), lambda i: (0, 0, 0)),
            pl.BlockSpec((1, 64), lambda i: (0, 0)),
            pl.BlockSpec((1, 64), lambda i: (0, 0)),
            pl.BlockSpec((64, 32), lambda i: (0, 0)),
        ],
        out_specs=pl.BlockSpec((BN, 32), lambda i: (i, 0)),
        out_shape=jax.ShapeDtypeStruct((NP, 32), jnp.float32),
    )(x1, ps, pss, g1r, b1r, w2)


def _k3a_body(pb, xw_ref, dinv_ref, b2_ref, x2_out, ps_out, pss_out):
    di = dinv_ref[...]
    x2 = pb[0] + pb[1] + xw_ref[...] * (di * di) + b2_ref[...]
    x2_out[...] = x2
    ps_out[...] = jnp.sum(x2, axis=0, keepdims=True)[None]
    pss_out[...] = jnp.sum(x2 * x2, axis=0, keepdims=True)[None]


def _k3a(pb, xw, dinv, b2r):
    return pl.pallas_call(
        _k3a_body,
        grid=(GRID_N,),
        in_specs=[
            pl.BlockSpec((2, BN, 32), lambda i: (0, i, 0)),
            pl.BlockSpec((BN, 32), lambda i: (i, 0)),
            pl.BlockSpec((BN, 1), lambda i: (i, 0)),
            pl.BlockSpec((1, 32), lambda i: (0, 0)),
        ],
        out_specs=[
            pl.BlockSpec((BN, 32), lambda i: (i, 0)),
            pl.BlockSpec((1, 1, 32), lambda i: (i, 0, 0)),
            pl.BlockSpec((1, 1, 32), lambda i: (i, 0, 0)),
        ],
        out_shape=[
            jax.ShapeDtypeStruct((N, 32), jnp.float32),
            jax.ShapeDtypeStruct((GRID_N, 1, 32), jnp.float32),
            jax.ShapeDtypeStruct((GRID_N, 1, 32), jnp.float32),
        ],
    )(pb, xw, dinv, b2r)


def _k3b_body(x2_ref, ps_ref, pss_ref, g_ref, b_ref, wf_ref, bf_ref, out_ref):
    S = jnp.sum(ps_ref[...], axis=0)
    SS = jnp.sum(pss_ref[...], axis=0)
    m = S / N
    v = SS / N - m * m
    sc = g_ref[...] * lax.rsqrt(v + 1e-5)
    sh = b_ref[...] - m * sc
    sc224 = jnp.concatenate([sc] * 7, axis=1)
    sh224 = jnp.concatenate([sh] * 7, axis=1)
    x2n = jnp.maximum(x2_ref[...] * sc224 + sh224, 0.0)
    out_ref[...] = (jnp.dot(x2n, wf_ref[...], preferred_element_type=jnp.float32)
                    + bf_ref[...])


def _k3b(x2r, ps, pss, g2r, b2r, wf, bfr):
    return pl.pallas_call(
        _k3b_body,
        grid=(5,),
        in_specs=[
            pl.BlockSpec((1000, 224), lambda i: (i, 0)),
            pl.BlockSpec((GRID_N, 1, 32), lambda i: (0, 0, 0)),
            pl.BlockSpec((GRID_N, 1, 32), lambda i: (0, 0, 0)),
            pl.BlockSpec((1, 32), lambda i: (0, 0)),
            pl.BlockSpec((1, 32), lambda i: (0, 0)),
            pl.BlockSpec((224, 8), lambda i: (0, 0)),
            pl.BlockSpec((1, 8), lambda i: (0, 0)),
        ],
        out_specs=pl.BlockSpec((1000, 8), lambda i: (i, 0)),
        out_shape=jax.ShapeDtypeStruct((5000, 8), jnp.float32),
    )(x2r, ps, pss, g2r, b2r, wf, bfr)


def kernel(x, edge_index, W1, a_src, a_dst, b1, gamma1, beta1,
           W2, b2, gamma2, beta2, Wf, bf):
    w1f = W1.reshape(128, 64)
    avs = a_src.reshape(1, 64)
    avd = a_dst.reshape(1, 64)

    h2, att4 = _k1(x, w1f, avs, avd)

    npad = NP - N
    h_st = h2.reshape(2 * NP, 32)
    att4p = jnp.pad(att4, ((0, npad), (0, 0)))
    att_sc = jnp.stack([jnp.stack([att4p[:, 0], att4p[:, 2]]),
                        jnp.stack([att4p[:, 1], att4p[:, 3]])])
    epad = EP - E
    src_pad = jnp.concatenate([edge_index[0], jnp.zeros((epad,), jnp.int32)])
    dst_pad = jnp.concatenate([edge_index[1], jnp.full((epad,), N, jnp.int32)])
    ei_pad = jnp.stack([src_pad, dst_pad])
    z32 = jnp.zeros((ROWS_PER_TILE, 32), jnp.float32)
    z48 = jnp.zeros((ROWS_PER_TILE, 48), jnp.float32)

    exh = _sc_att(ei_pad, att_sc)
    agg48 = _sc_agg(ei_pad, h_st, exh, z48)

    b1r = b1.reshape(1, 64)
    x1, ps1, pss1, dinv = _k2a(h2, att4, agg48, b1r)
    xw = _k2b(x1, ps1, pss1, gamma1.reshape(1, 64), beta1.reshape(1, 64), W2)

    outB = _sc_gcn(ei_pad, xw, dinv.reshape(NP), z32)

    x2, ps2, pss2 = _k3a(outB, xw, dinv, b2.reshape(1, 32))
    out = _k3b(x2.reshape(5000, 224), ps2, pss2,
               gamma2.reshape(1, 32), beta2.reshape(1, 32), Wf,
               bf.reshape(1, 8))
    return out
